# baseline (speedup 1.0000x reference)
import functools

import jax
import jax.numpy as jnp
from jax import lax
from jax.experimental import pallas as pl
from jax.experimental.pallas import tpu as pltpu
from jax.experimental.pallas import tpu_sc as plsc

ROWS, SEQ = 16384, 50
CATEGORY, DIST_EMBED = 64, 32
B = ROWS * SEQ
NW = 32
BW = B // NW
C = 1024
NCHUNK = BW // C
L = 16
G = 128
NG = C // G

_mesh = plsc.VectorSubcoreMesh(core_axis_name="c", subcore_axis_name="s")


@functools.partial(
    pl.kernel,
    mesh=_mesh,
    compiler_params=pltpu.CompilerParams(
        needs_layout_passes=False, use_tc_tiling_on_sc=False
    ),
    out_type=jax.ShapeDtypeStruct((B, DIST_EMBED), jnp.float32),
    scratch_types=[
        pltpu.VMEM((C,), jnp.int32),
        pltpu.VMEM((C,), jnp.int32),
        pltpu.VMEM((C,), jnp.int32),
        pltpu.VMEM((C, DIST_EMBED), jnp.float32),
        pltpu.SemaphoreType.DMA,
    ],
)
def _lookup(span_hbm, np_hbm, table_hbm, out_hbm, a_v, b_v, d_v, rows_v, sem):
    wid = lax.axis_index("s") * 2 + lax.axis_index("c")
    base = wid * BW

    def chunk_body(ci, carry):
        off = base + ci * C
        pltpu.sync_copy(span_hbm.at[pl.ds(off, C)], a_v)
        pltpu.sync_copy(np_hbm.at[pl.ds(off, C)], b_v)

        def grp(j, c2):
            a = a_v[pl.ds(j * L, L)]
            b = b_v[pl.ds(j * L, L)]
            d_v[pl.ds(j * L, L)] = jnp.minimum(jnp.abs(a - b), CATEGORY - 1)
            return c2

        lax.fori_loop(0, C // L, grp, 0)

        copies = [
            pltpu.async_copy(
                table_hbm.at[d_v.at[pl.ds(g * G, G)]],
                rows_v.at[pl.ds(g * G, G)],
                sem,
            )
            for g in range(NG)
        ]
        for c in copies:
            c.wait()
        pltpu.sync_copy(rows_v, out_hbm.at[pl.ds(off, C)])
        return carry

    lax.fori_loop(0, NCHUNK, chunk_body, 0)


def kernel(span_sentence_index, np_sentence_index, distance_embeddings):
    span = span_sentence_index.reshape(-1)
    npi = np_sentence_index.reshape(-1)
    out = _lookup(span, npi, distance_embeddings)
    return out.reshape(ROWS, SEQ, DIST_EMBED)


# speedup vs baseline: 7.2230x; 7.2230x over previous
import functools

import jax
import jax.numpy as jnp
from jax import lax
from jax.experimental import pallas as pl
from jax.experimental.pallas import tpu as pltpu
from jax.experimental.pallas import tpu_sc as plsc

ROWS, SEQ = 16384, 50
CATEGORY, DIST_EMBED = 64, 32
B = ROWS * SEQ
NW = 32
BW = B // NW
C = 1280
NCHUNK = BW // C
L = 16

_mesh = plsc.VectorSubcoreMesh(core_axis_name="c", subcore_axis_name="s")


@functools.partial(
    pl.kernel,
    mesh=_mesh,
    compiler_params=pltpu.CompilerParams(needs_layout_passes=False),
    out_type=jax.ShapeDtypeStruct((B * DIST_EMBED,), jnp.float32),
    scratch_types=[
        pltpu.VMEM((CATEGORY * DIST_EMBED,), jnp.float32),
        pltpu.VMEM((C,), jnp.int32),
        pltpu.VMEM((C,), jnp.int32),
        pltpu.VMEM((C * DIST_EMBED,), jnp.float32),
        pltpu.VMEM((C * DIST_EMBED,), jnp.float32),
        pltpu.SemaphoreType.DMA,
        pltpu.SemaphoreType.DMA,
    ],
)
def _lookup(span_hbm, np_hbm, table_hbm, out_hbm, table_v,
            a_v, b_v, r0, r1, sem_out0, sem_out1):
    wid = lax.axis_index("s") * 2 + lax.axis_index("c")
    base = wid * BW
    pltpu.sync_copy(table_hbm, table_v)
    r_bufs = (r0, r1)
    sem_out = (sem_out0, sem_out1)

    def compute_chunk(ci, rk):
        off = base + ci * C
        pltpu.sync_copy(span_hbm.at[pl.ds(off, C)], a_v)
        pltpu.sync_copy(np_hbm.at[pl.ds(off, C)], b_v)

        @plsc.parallel_loop(0, C // L, unroll=2)
        def grp(j):
            a = a_v[pl.ds(j * L, L)]
            b = b_v[pl.ds(j * L, L)]
            d = jnp.minimum(jnp.abs(a - b), CATEGORY - 1)
            g = d * DIST_EMBED
            s = (lax.iota(jnp.int32, L) + j * L) * DIST_EMBED
            for col in range(DIST_EMBED):
                vals = plsc.load_gather(table_v, [g + col])
                plsc.store_scatter(rk, [s + col], vals)

        return pltpu.async_copy(
            rk, out_hbm.at[pl.ds(off * DIST_EMBED, C * DIST_EMBED)], sem_out[0 if rk is r0 else 1]
        )

    compute_chunk(0, r0)
    compute_chunk(1, r1)

    def pair_body(i, carry):
        for k in range(2):
            ci = 2 + i * 2 + k
            rk = r_bufs[k]
            pltpu.make_async_copy(
                rk, out_hbm.at[pl.ds((base + ci * C) * DIST_EMBED, C * DIST_EMBED)],
                sem_out[k],
            ).wait()
            compute_chunk(ci, rk)
        return carry

    lax.fori_loop(0, (NCHUNK - 2) // 2, pair_body, 0)

    for k in range(2):
        pltpu.make_async_copy(
            r_bufs[k], out_hbm.at[pl.ds(base * DIST_EMBED, C * DIST_EMBED)], sem_out[k]
        ).wait()


def kernel(span_sentence_index, np_sentence_index, distance_embeddings):
    span = span_sentence_index.reshape(-1)
    npi = np_sentence_index.reshape(-1)
    out = _lookup(span, npi, distance_embeddings.reshape(-1))
    return out.reshape(ROWS, SEQ, DIST_EMBED)
